# baseline (speedup 1.0000x reference)
import jax
import jax.numpy as jnp
from jax import lax
from jax.experimental import pallas as pl
from jax.experimental.pallas import tpu as pltpu
from jax.experimental.pallas import tpu_sc as plsc

_B = 4
_P = 256
_D = 1024
_S = 2048
_NW = 32
_RPW = _P // _NW


def _body(ids_hbm, pids_hbm, table_hbm, out_ids, out_emb,
          idx_v, rows_v, ids_v, pids_v, gsem, wsem, csem, osem):
    c = lax.axis_index("c")
    s = lax.axis_index("s")
    wid = s * 2 + c
    pbase = wid * _RPW

    pltpu.sync_copy(pids_hbm.at[pl.ds(pbase, _RPW)], idx_v)
    gather = pltpu.make_async_copy(table_hbm.at[idx_v], rows_v, gsem)
    gather.start()

    @pl.when(wid < _B)
    def _():
        pltpu.make_async_copy(ids_hbm.at[wid], ids_v, csem).start()

    @pl.when(jnp.logical_and(wid >= _B, wid < 2 * _B))
    def _():
        pltpu.make_async_copy(pids_hbm, pids_v, csem).start()

    gather.wait()
    writes = []
    for b in range(_B):
        w = pltpu.make_async_copy(
            rows_v, out_emb.at[pl.ds(b * _P + pbase, _RPW)], wsem)
        w.start()
        writes.append(w)

    @pl.when(wid < _B)
    def _():
        pltpu.make_async_copy(ids_hbm.at[wid], ids_v, csem).wait()
        pltpu.make_async_copy(ids_v, out_ids.at[wid, pl.ds(_P, _S)], osem).start()

    @pl.when(jnp.logical_and(wid >= _B, wid < 2 * _B))
    def _():
        pltpu.make_async_copy(pids_hbm, pids_v, csem).wait()
        pltpu.make_async_copy(pids_v, out_ids.at[wid - _B, pl.ds(0, _P)], osem).start()

    for w in writes:
        w.wait()

    @pl.when(wid < _B)
    def _():
        pltpu.make_async_copy(ids_v, out_ids.at[wid, pl.ds(_P, _S)], osem).wait()

    @pl.when(jnp.logical_and(wid >= _B, wid < 2 * _B))
    def _():
        pltpu.make_async_copy(pids_v, out_ids.at[wid - _B, pl.ds(0, _P)], osem).wait()


@jax.jit
def _sc_call(input_ids, prompt_ids, embed_weight):
    mesh = plsc.VectorSubcoreMesh(core_axis_name="c", subcore_axis_name="s")
    f = pl.kernel(
        _body,
        mesh=mesh,
        out_type=(
            jax.ShapeDtypeStruct((_B, _P + _S), jnp.int32),
            jax.ShapeDtypeStruct((_B * _P, _D), jnp.float32),
        ),
        scratch_types=[
            pltpu.VMEM((_RPW,), jnp.int32),
            pltpu.VMEM((_RPW, _D), jnp.float32),
            pltpu.VMEM((_S,), jnp.int32),
            pltpu.VMEM((_P,), jnp.int32),
            pltpu.SemaphoreType.DMA,
            pltpu.SemaphoreType.DMA,
            pltpu.SemaphoreType.DMA,
            pltpu.SemaphoreType.DMA,
        ],
    )
    return f(input_ids, prompt_ids, embed_weight)


def kernel(input_ids, prompt_ids, embed_weight):
    out_ids, emb = _sc_call(input_ids, prompt_ids, embed_weight)
    return out_ids, emb.reshape(_B, _P, _D)
